# baseline (speedup 1.0000x reference)
import functools
import math

import jax
import jax.numpy as jnp
from jax import lax
from jax.experimental import pallas as pl
from jax.experimental.pallas import tpu as pltpu
from jax.experimental.pallas import tpu_sc as plsc

MODEL_DIM = 64
SCALE = math.sqrt(MODEL_DIM)
LANES = 16
NUM_WORKERS = 32
NG = 4
NS = 2
ROW_UNROLL = 4


def _emb_kernel_body(n_batches, seq, idx_hbm, w_hbm, out_hbm, idx_v, rows_g,
                     rows_s, gs0, gs1, gs2, gs3, ss0, ss1):
    b_per_w = n_batches * seq
    gsems = (gs0, gs1, gs2, gs3)
    ssems = (ss0, ss1)
    cid = lax.axis_index("c")
    sid = lax.axis_index("s")
    wid = sid * 2 + cid
    base = wid * b_per_w
    batch0 = wid * n_batches
    pltpu.sync_copy(idx_hbm.at[pl.ds(base, b_per_w)], idx_v)

    def gather_cps(b, buf):
        off = b * seq
        return (
            pltpu.make_async_copy(
                w_hbm.at[idx_v.at[pl.ds(off, 128)]],
                rows_g.at[buf, pl.ds(0, 128)], gsems[buf]),
            pltpu.make_async_copy(
                w_hbm.at[idx_v.at[pl.ds(off + 128, seq - 128)]],
                rows_g.at[buf, pl.ds(128, seq - 128)], gsems[buf]),
        )

    def store_cp(b, sb):
        dst = out_hbm.at[batch0 + b]
        return pltpu.make_async_copy(rows_s.at[sb], dst, ssems[sb])

    for buf in range(NG):
        for cp in gather_cps(buf, buf):
            cp.start()

    def group_body(g, carry):
        for buf in range(NG):
            b = g * NG + buf
            sb = buf % NS
            if buf < NS:
                @pl.when(g > 0)
                def _wait_prev_store():
                    store_cp(b - NS, sb).wait()
            else:
                store_cp(b - NS, sb).wait()
            for cp in gather_cps(b, buf):
                cp.wait()
            rg = rows_g.at[buf]
            rs = rows_s.at[sb]

            @plsc.parallel_loop(0, seq, unroll=ROW_UNROLL)
            def _row_body(r):
                for cc in range(MODEL_DIM // LANES):
                    sl = pl.ds(cc * LANES, LANES)
                    rs[r, sl] = rg[r, sl] * SCALE

            store_cp(b, sb).start()
            @pl.when(b + NG < n_batches)
            def _refill():
                for cp in gather_cps(b + NG, buf):
                    cp.start()
        return carry

    lax.fori_loop(0, n_batches // NG, group_body, 0)
    for k in range(NS):
        b = n_batches - NS + k
        store_cp(b, b % NS).wait()


def kernel(input_ids, weight):
    n_rows, seq = input_ids.shape
    total = n_rows * seq
    n_batches = n_rows // NUM_WORKERS
    assert n_batches * NUM_WORKERS == n_rows and n_batches % NG == 0
    assert 128 < seq <= 256 and seq % 8 == 0

    idx = input_ids.reshape(total).astype(jnp.int32)

    mesh = plsc.VectorSubcoreMesh(core_axis_name="c", subcore_axis_name="s")
    emb = functools.partial(
        pl.kernel,
        mesh=mesh,
        out_type=jax.ShapeDtypeStruct((n_rows, seq, MODEL_DIM), jnp.float32),
        scratch_types=[
            pltpu.VMEM((n_batches * seq,), jnp.int32),
            pltpu.VMEM((NG, seq, MODEL_DIM), jnp.float32),
            pltpu.VMEM((NS, seq, MODEL_DIM), jnp.float32),
        ] + [pltpu.SemaphoreType.DMA] * (NG + NS),
        compiler_params=pltpu.CompilerParams(use_tc_tiling_on_sc=False),
    )(functools.partial(_emb_kernel_body, n_batches, seq))

    return emb(idx, weight)


# speedup vs baseline: 1.0033x; 1.0033x over previous
import functools
import math

import jax
import jax.numpy as jnp
from jax import lax
from jax.experimental import pallas as pl
from jax.experimental.pallas import tpu as pltpu
from jax.experimental.pallas import tpu_sc as plsc

MODEL_DIM = 64
SCALE = math.sqrt(MODEL_DIM)
LANES = 16
NUM_WORKERS = 32
NG = 4
NS = 2
ROW_UNROLL = 4


def _emb_kernel_body(n_batches, seq, idx_hbm, w_hbm, out_hbm, idx_v, rows_g,
                     rows_s, gs0, gs1, gs2, gs3, ss0, ss1):
    b_per_w = n_batches * seq
    gsems = (gs0, gs1, gs2, gs3)
    ssems = (ss0, ss1)
    cid = lax.axis_index("c")
    sid = lax.axis_index("s")
    wid = sid * 2 + cid
    base = wid * b_per_w
    batch0 = wid * n_batches
    pltpu.sync_copy(idx_hbm.at[pl.ds(base, b_per_w)], idx_v)

    def gather_cps(b, buf):
        off = b * seq
        return (
            pltpu.make_async_copy(
                w_hbm.at[idx_v.at[pl.ds(off, 128)]],
                rows_g.at[buf, pl.ds(0, 128)], gsems[buf]),
            pltpu.make_async_copy(
                w_hbm.at[idx_v.at[pl.ds(off + 128, seq - 128)]],
                rows_g.at[buf, pl.ds(128, seq - 128)], gsems[buf]),
        )

    def store_cp(b, sb):
        dst = out_hbm.at[batch0 + b]
        return pltpu.make_async_copy(rows_s.at[sb], dst, ssems[sb])

    for buf in range(NG):
        for cp in gather_cps(buf, buf):
            cp.start()

    def group_body(g, carry):
        for buf in range(NG):
            b = g * NG + buf
            sb = buf % NS
            if buf < NS:
                @pl.when(g > 0)
                def _wait_prev_store():
                    store_cp(b - NS, sb).wait()
            else:
                store_cp(b - NS, sb).wait()
            for cp in gather_cps(b, buf):
                cp.wait()
            rg = rows_g.at[buf]
            rs = rows_s.at[sb]

            @plsc.parallel_loop(0, seq, unroll=ROW_UNROLL)
            def _row_body(r):
                for cc in range(MODEL_DIM // LANES):
                    sl = pl.ds(cc * LANES, LANES)
                    rs[r, sl] = rg[r, sl] * SCALE

            store_cp(b, sb).start()
            @pl.when(b + NG < n_batches)
            def _refill():
                for cp in gather_cps(b + NG, buf):
                    cp.start()
        return carry

    lax.fori_loop(0, n_batches // NG, group_body, 0)
    for k in range(NS):
        b = n_batches - NS + k
        store_cp(b, b % NS).wait()


def kernel(input_ids, weight):
    n_rows, seq = input_ids.shape
    total = n_rows * seq
    n_batches = n_rows // NUM_WORKERS
    assert n_batches * NUM_WORKERS == n_rows and n_batches % NG == 0
    assert 128 < seq <= 256 and seq % 8 == 0

    idx = input_ids.reshape(total).astype(jnp.int32)

    mesh = plsc.VectorSubcoreMesh(core_axis_name="c", subcore_axis_name="s")
    emb = functools.partial(
        pl.kernel,
        mesh=mesh,
        out_type=jax.ShapeDtypeStruct((n_rows, seq, MODEL_DIM), jnp.float32),
        scratch_types=[
            pltpu.VMEM((n_batches * seq,), jnp.int32),
            pltpu.VMEM((NG, seq, MODEL_DIM), jnp.float32),
            pltpu.VMEM((NS, seq, MODEL_DIM), jnp.float32),
        ] + [pltpu.SemaphoreType.DMA] * (NG + NS),
        compiler_params=pltpu.CompilerParams(
            use_tc_tiling_on_sc=False, needs_layout_passes=False),
    )(functools.partial(_emb_kernel_body, n_batches, seq))

    return emb(idx, weight)
